# Initial kernel scaffold; baseline (speedup 1.0000x reference)
import jax
import jax.numpy as jnp
from jax.experimental import pallas as pl


def kernel(image_shape, boxes, scores, p0, p1, p2, p3, p4):
    raise NotImplementedError("write your pallas kernel here")


# speedup vs baseline: 10.9644x; 10.9644x over previous
import functools

import jax
import jax.numpy as jnp
from jax import lax
from jax.experimental import pallas as pl
from jax.experimental.pallas import tpu as pltpu

CROP = 14
C = 256
LEVEL_HW = [(64, 64), (32, 32), (16, 16), (8, 8), (4, 4)]
HMAX, WMAX = 64, 64
EPS = 1e-7
YPAD = HMAX + 1
KX = sum(w for _, w in LEVEL_HW)
NCOL = YPAD * C


def _tc_body(nb_blk, y0_sm, wy0_sm, wy1_sm, sx0_ref, sx1_ref, wx0_ref,
             wx1_ref, tab_ref, out_ref, ta_ref):
    i = pl.program_id(0)
    m = nb_blk * CROP
    sx0 = sx0_ref[...]
    sx1 = sx1_ref[...]
    wx0 = wx0_ref[...]
    wx1 = wx1_ref[...]
    iota = lax.broadcasted_iota(jnp.int32, (m, KX), 1)
    w_oh = (jnp.where(iota == sx0, wx0, 0.0)
            + jnp.where(iota == sx1, wx1, 0.0))
    ta_ref[...] = jnp.dot(w_oh, tab_ref[...],
                          preferred_element_type=jnp.float32).reshape(
                              nb_blk, CROP, NCOL)

    def body(bj, _):
        b = bj // CROP
        j = bj - b * CROP
        g = i * m + bj
        y0 = y0_sm[g]
        ystart = pl.multiple_of(y0 * C, C)
        s = ta_ref[b, :, pl.ds(ystart, 2 * C)]
        out_ref[b, j] = s[:, :C] * wy0_sm[g] + s[:, C:] * wy1_sm[g]
        return 0

    lax.fori_loop(0, m, body, 0)


def _run(nb, bb, image_shape, boxes, p_list):
    img = image_shape.astype(jnp.float32)
    b = boxes[0]
    x1, y1, x2, y2 = b[:, 0], b[:, 1], b[:, 2], b[:, 3]
    w = x2 - x1
    h = y2 - y1
    size = jnp.sqrt(w * h)
    levels = jnp.floor(1.0 + jnp.log2(size / 224.0 + EPS))
    levels = jnp.clip(levels, 0.0, 4.0).astype(jnp.int32)
    Hs = jnp.array([hw[0] for hw in LEVEL_HW], dtype=jnp.float32)
    Ws = jnp.array([hw[1] for hw in LEVEL_HW], dtype=jnp.float32)
    fh = Hs[levels]
    fw = Ws[levels]
    y1n = y1 / img[1] * fh / (fh - 1.0)
    x1n = x1 / img[2] * fw / (fw - 1.0)
    y2n = (y2 / img[1] * fh - 1.0) / (fh - 1.0)
    x2n = (x2 / img[2] * fw - 1.0) / (fw - 1.0)
    i14 = jnp.arange(CROP, dtype=jnp.float32)
    ys = (y1n[:, None] * (fh[:, None] - 1.0)
          + i14[None, :] * (y2n - y1n)[:, None] * (fh[:, None] - 1.0) / (CROP - 1.0))
    xs = (x1n[:, None] * (fw[:, None] - 1.0)
          + i14[None, :] * (x2n - x1n)[:, None] * (fw[:, None] - 1.0) / (CROP - 1.0))
    valid_y = ((ys >= 0.0) & (ys <= fh[:, None] - 1.0)).astype(jnp.float32)
    valid_x = ((xs >= 0.0) & (xs <= fw[:, None] - 1.0)).astype(jnp.float32)
    y0f = jnp.floor(ys)
    x0f = jnp.floor(xs)
    ly = ys - y0f
    lx = xs - x0f
    y0i = jnp.clip(y0f, 0, HMAX - 1).astype(jnp.int32)
    x0i = jnp.clip(x0f, 0, WMAX - 1).astype(jnp.int32)
    x1i = jnp.clip(x0f + 1.0, 0, WMAX - 1).astype(jnp.int32)
    xoffs = []
    acc = 0
    for _, wl in LEVEL_HW:
        xoffs.append(acc)
        acc += wl
    xoffv = jnp.array(xoffs, dtype=jnp.int32)[levels]
    sx0 = xoffv[:, None] + x0i
    sx1 = xoffv[:, None] + x1i
    wy0 = (1.0 - ly) * valid_y
    wy1 = ly * valid_y
    wx0 = (1.0 - lx) * valid_x
    wx1 = lx * valid_x

    rows = []
    for l, (hl, wl) in enumerate(LEVEL_HW):
        f = p_list[l][0]
        ft = jnp.transpose(f, (1, 0, 2))
        ft = jnp.pad(ft, ((0, 0), (0, YPAD - hl), (0, 0)))
        rows.append(ft.reshape(wl, NCOL))
    tab = jnp.concatenate(rows, axis=0)

    n_blk = nb // bb
    m = bb * CROP
    col = lambda a: a.reshape(nb * CROP, 1)

    grid_spec = pltpu.PrefetchScalarGridSpec(
        num_scalar_prefetch=3,
        grid=(n_blk,),
        in_specs=[
            pl.BlockSpec((m, 1), lambda i, *_: (i, 0)),
            pl.BlockSpec((m, 1), lambda i, *_: (i, 0)),
            pl.BlockSpec((m, 1), lambda i, *_: (i, 0)),
            pl.BlockSpec((m, 1), lambda i, *_: (i, 0)),
            pl.BlockSpec((KX, NCOL), lambda i, *_: (0, 0)),
        ],
        out_specs=pl.BlockSpec((bb, CROP, CROP, C), lambda i, *_: (i, 0, 0, 0)),
        scratch_shapes=[pltpu.VMEM((bb, CROP, NCOL), jnp.float32)],
    )
    out4 = pl.pallas_call(
        functools.partial(_tc_body, bb),
        grid_spec=grid_spec,
        out_shape=jax.ShapeDtypeStruct((nb, CROP, CROP, C), jnp.float32),
    )(y0i.reshape(-1), wy0.reshape(-1), wy1.reshape(-1),
      col(sx0), col(sx1), col(wx0), col(wx1), tab)
    return out4[None]


def kernel(image_shape, boxes, scores, p0, p1, p2, p3, p4):
    del scores
    return _run(boxes.shape[1], 8, image_shape, boxes, (p0, p1, p2, p3, p4))


# speedup vs baseline: 14.4218x; 1.3153x over previous
import functools

import jax
import jax.numpy as jnp
from jax import lax
from jax.experimental import pallas as pl
from jax.experimental.pallas import tpu as pltpu

CROP = 14
C = 256
LEVEL_HW = [(64, 64), (32, 32), (16, 16), (8, 8), (4, 4)]
HMAX, WMAX = 64, 64
EPS = 1e-7
YPAD = HMAX + 1
KX = sum(w for _, w in LEVEL_HW)
NCOL = YPAD * C


ROWPAD = 16


def _tc_body(nb_blk, y0_sm, wy0_sm, wy1_sm, sx0_ref, sx1_ref, wx0_ref,
             wx1_ref, tab_ref, out_ref, ta_ref):
    i = pl.program_id(0)
    mp = nb_blk * ROWPAD
    sx0 = sx0_ref[...]
    sx1 = sx1_ref[...]
    wx0 = wx0_ref[...]
    wx1 = wx1_ref[...]
    iota = lax.broadcasted_iota(jnp.int32, (mp, KX), 1)
    w_oh = (jnp.where(iota == sx0, wx0, 0.0)
            + jnp.where(iota == sx1, wx1, 0.0))
    ta_ref[...] = jnp.dot(w_oh, tab_ref[...],
                          preferred_element_type=jnp.float32)

    for b in range(nb_blk):
        for j in range(CROP):
            g = i * (nb_blk * CROP) + b * CROP + j
            y0 = y0_sm[g]
            ystart = pl.multiple_of(y0 * C, C)
            s = ta_ref[pl.ds(b * ROWPAD, CROP), pl.ds(ystart, 2 * C)]
            out_ref[b, j] = s[:, :C] * wy0_sm[g] + s[:, C:] * wy1_sm[g]


def _run(nb, bb, image_shape, boxes, p_list):
    img = image_shape.astype(jnp.float32)
    b = boxes[0]
    x1, y1, x2, y2 = b[:, 0], b[:, 1], b[:, 2], b[:, 3]
    w = x2 - x1
    h = y2 - y1
    size = jnp.sqrt(w * h)
    levels = jnp.floor(1.0 + jnp.log2(size / 224.0 + EPS))
    levels = jnp.clip(levels, 0.0, 4.0).astype(jnp.int32)
    Hs = jnp.array([hw[0] for hw in LEVEL_HW], dtype=jnp.float32)
    Ws = jnp.array([hw[1] for hw in LEVEL_HW], dtype=jnp.float32)
    fh = Hs[levels]
    fw = Ws[levels]
    y1n = y1 / img[1] * fh / (fh - 1.0)
    x1n = x1 / img[2] * fw / (fw - 1.0)
    y2n = (y2 / img[1] * fh - 1.0) / (fh - 1.0)
    x2n = (x2 / img[2] * fw - 1.0) / (fw - 1.0)
    i14 = jnp.arange(CROP, dtype=jnp.float32)
    ys = (y1n[:, None] * (fh[:, None] - 1.0)
          + i14[None, :] * (y2n - y1n)[:, None] * (fh[:, None] - 1.0) / (CROP - 1.0))
    xs = (x1n[:, None] * (fw[:, None] - 1.0)
          + i14[None, :] * (x2n - x1n)[:, None] * (fw[:, None] - 1.0) / (CROP - 1.0))
    valid_y = ((ys >= 0.0) & (ys <= fh[:, None] - 1.0)).astype(jnp.float32)
    valid_x = ((xs >= 0.0) & (xs <= fw[:, None] - 1.0)).astype(jnp.float32)
    y0f = jnp.floor(ys)
    x0f = jnp.floor(xs)
    ly = ys - y0f
    lx = xs - x0f
    y0i = jnp.clip(y0f, 0, HMAX - 1).astype(jnp.int32)
    x0i = jnp.clip(x0f, 0, WMAX - 1).astype(jnp.int32)
    x1i = jnp.clip(x0f + 1.0, 0, WMAX - 1).astype(jnp.int32)
    xoffs = []
    acc = 0
    for _, wl in LEVEL_HW:
        xoffs.append(acc)
        acc += wl
    xoffv = jnp.array(xoffs, dtype=jnp.int32)[levels]
    sx0 = xoffv[:, None] + x0i
    sx1 = xoffv[:, None] + x1i
    wy0 = (1.0 - ly) * valid_y
    wy1 = ly * valid_y
    wx0 = (1.0 - lx) * valid_x
    wx1 = lx * valid_x

    rows = []
    for l, (hl, wl) in enumerate(LEVEL_HW):
        f = p_list[l][0]
        ft = jnp.transpose(f, (1, 0, 2))
        ft = jnp.pad(ft, ((0, 0), (0, YPAD - hl), (0, 0)))
        rows.append(ft.reshape(wl, NCOL))
    tab = jnp.concatenate(rows, axis=0)

    n_blk = nb // bb
    mp = bb * ROWPAD
    padrows = ((0, 0), (0, ROWPAD - CROP))
    coli = lambda a: jnp.pad(a, padrows, constant_values=-1).reshape(
        nb * ROWPAD, 1)
    colf = lambda a: jnp.pad(a, padrows).reshape(nb * ROWPAD, 1)

    grid_spec = pltpu.PrefetchScalarGridSpec(
        num_scalar_prefetch=3,
        grid=(n_blk,),
        in_specs=[
            pl.BlockSpec((mp, 1), lambda i, *_: (i, 0)),
            pl.BlockSpec((mp, 1), lambda i, *_: (i, 0)),
            pl.BlockSpec((mp, 1), lambda i, *_: (i, 0)),
            pl.BlockSpec((mp, 1), lambda i, *_: (i, 0)),
            pl.BlockSpec((KX, NCOL), lambda i, *_: (0, 0)),
        ],
        out_specs=pl.BlockSpec((bb, CROP, CROP, C), lambda i, *_: (i, 0, 0, 0)),
        scratch_shapes=[pltpu.VMEM((mp, NCOL), jnp.float32)],
    )
    out4 = pl.pallas_call(
        functools.partial(_tc_body, bb),
        grid_spec=grid_spec,
        out_shape=jax.ShapeDtypeStruct((nb, CROP, CROP, C), jnp.float32),
    )(y0i.reshape(-1), wy0.reshape(-1), wy1.reshape(-1),
      coli(sx0), coli(sx1), colf(wx0), colf(wx1), tab)
    return out4[None]


def kernel(image_shape, boxes, scores, p0, p1, p2, p3, p4):
    del scores
    return _run(boxes.shape[1], 8, image_shape, boxes, (p0, p1, p2, p3, p4))


# speedup vs baseline: 14.6460x; 1.0155x over previous
import functools

import jax
import jax.numpy as jnp
from jax import lax
from jax.experimental import pallas as pl
from jax.experimental.pallas import tpu as pltpu

CROP = 14
C = 256
LEVEL_HW = [(64, 64), (32, 32), (16, 16), (8, 8), (4, 4)]
HMAX, WMAX = 64, 64
EPS = 1e-7
YPAD = HMAX + 1
KX = sum(w for _, w in LEVEL_HW)
NCOL = YPAD * C


ROWPAD = 16
XOFFS = [0, 64, 96, 112, 120]


def _tc_body(nb_blk, y0_sm, wy0_sm, wy1_sm, sx0_ref, sx1_ref, wx0_ref,
             wx1_ref, p0_ref, p1_ref, p2_ref, p3_ref, p4_ref,
             out_ref, ta_ref, tab_ref):
    i = pl.program_id(0)
    mp = nb_blk * ROWPAD

    @pl.when(i == 0)
    def _build():
        for lvl, p_ref in enumerate((p0_ref, p1_ref, p2_ref, p3_ref, p4_ref)):
            hl, wl = LEVEL_HW[lvl]
            xo = XOFFS[lvl]
            for y in range(hl):
                tab_ref[pl.ds(xo, wl), pl.ds(y * C, C)] = p_ref[y]
            tab_ref[pl.ds(xo, wl), pl.ds(hl * C, (YPAD - hl) * C)] = (
                jnp.zeros((wl, (YPAD - hl) * C), jnp.float32))
    sx0 = sx0_ref[...]
    sx1 = sx1_ref[...]
    wx0 = wx0_ref[...]
    wx1 = wx1_ref[...]
    iota = lax.broadcasted_iota(jnp.int32, (mp, KX), 1)
    w_oh = (jnp.where(iota == sx0, wx0, 0.0)
            + jnp.where(iota == sx1, wx1, 0.0))
    ta_ref[...] = jnp.dot(w_oh, tab_ref[...],
                          preferred_element_type=jnp.float32)

    for b in range(nb_blk):
        for j in range(CROP):
            g = i * (nb_blk * CROP) + b * CROP + j
            y0 = y0_sm[g]
            ystart = pl.multiple_of(y0 * C, C)
            s = ta_ref[pl.ds(b * ROWPAD, CROP), pl.ds(ystart, 2 * C)]
            out_ref[b, j] = s[:, :C] * wy0_sm[g] + s[:, C:] * wy1_sm[g]


def _run(nb, bb, image_shape, boxes, p_list):
    img = image_shape.astype(jnp.float32)
    b = boxes[0]
    x1, y1, x2, y2 = b[:, 0], b[:, 1], b[:, 2], b[:, 3]
    w = x2 - x1
    h = y2 - y1
    size = jnp.sqrt(w * h)
    levels = jnp.floor(1.0 + jnp.log2(size / 224.0 + EPS))
    levels = jnp.clip(levels, 0.0, 4.0).astype(jnp.int32)
    Hs = jnp.array([hw[0] for hw in LEVEL_HW], dtype=jnp.float32)
    Ws = jnp.array([hw[1] for hw in LEVEL_HW], dtype=jnp.float32)
    fh = Hs[levels]
    fw = Ws[levels]
    y1n = y1 / img[1] * fh / (fh - 1.0)
    x1n = x1 / img[2] * fw / (fw - 1.0)
    y2n = (y2 / img[1] * fh - 1.0) / (fh - 1.0)
    x2n = (x2 / img[2] * fw - 1.0) / (fw - 1.0)
    i14 = jnp.arange(CROP, dtype=jnp.float32)
    ys = (y1n[:, None] * (fh[:, None] - 1.0)
          + i14[None, :] * (y2n - y1n)[:, None] * (fh[:, None] - 1.0) / (CROP - 1.0))
    xs = (x1n[:, None] * (fw[:, None] - 1.0)
          + i14[None, :] * (x2n - x1n)[:, None] * (fw[:, None] - 1.0) / (CROP - 1.0))
    valid_y = ((ys >= 0.0) & (ys <= fh[:, None] - 1.0)).astype(jnp.float32)
    valid_x = ((xs >= 0.0) & (xs <= fw[:, None] - 1.0)).astype(jnp.float32)
    y0f = jnp.floor(ys)
    x0f = jnp.floor(xs)
    ly = ys - y0f
    lx = xs - x0f
    y0i = jnp.clip(y0f, 0, HMAX - 1).astype(jnp.int32)
    x0i = jnp.clip(x0f, 0, WMAX - 1).astype(jnp.int32)
    x1i = jnp.clip(x0f + 1.0, 0, WMAX - 1).astype(jnp.int32)
    xoffv = jnp.array(XOFFS, dtype=jnp.int32)[levels]
    sx0 = xoffv[:, None] + x0i
    sx1 = xoffv[:, None] + x1i
    wy0 = (1.0 - ly) * valid_y
    wy1 = ly * valid_y
    wx0 = (1.0 - lx) * valid_x
    wx1 = lx * valid_x

    n_blk = nb // bb
    mp = bb * ROWPAD
    padrows = ((0, 0), (0, ROWPAD - CROP))
    coli = lambda a: jnp.pad(a, padrows, constant_values=-1).reshape(
        nb * ROWPAD, 1)
    colf = lambda a: jnp.pad(a, padrows).reshape(nb * ROWPAD, 1)

    grid_spec = pltpu.PrefetchScalarGridSpec(
        num_scalar_prefetch=3,
        grid=(n_blk,),
        in_specs=[
            pl.BlockSpec((mp, 1), lambda i, *_: (i, 0)),
            pl.BlockSpec((mp, 1), lambda i, *_: (i, 0)),
            pl.BlockSpec((mp, 1), lambda i, *_: (i, 0)),
            pl.BlockSpec((mp, 1), lambda i, *_: (i, 0)),
        ] + [
            pl.BlockSpec(
                (LEVEL_HW[l][0], LEVEL_HW[l][1], C),
                lambda i, *_: (0, 0, 0))
            for l in range(5)
        ],
        out_specs=pl.BlockSpec((bb, CROP, CROP, C), lambda i, *_: (i, 0, 0, 0)),
        scratch_shapes=[pltpu.VMEM((mp, NCOL), jnp.float32),
                        pltpu.VMEM((KX, NCOL), jnp.float32)],
    )
    out4 = pl.pallas_call(
        functools.partial(_tc_body, bb),
        grid_spec=grid_spec,
        out_shape=jax.ShapeDtypeStruct((nb, CROP, CROP, C), jnp.float32),
    )(y0i.reshape(-1), wy0.reshape(-1), wy1.reshape(-1),
      coli(sx0), coli(sx1), colf(wx0), colf(wx1),
      p_list[0][0], p_list[1][0], p_list[2][0], p_list[3][0], p_list[4][0])
    return out4[None]


def kernel(image_shape, boxes, scores, p0, p1, p2, p3, p4):
    del scores
    return _run(boxes.shape[1], 8, image_shape, boxes, (p0, p1, p2, p3, p4))


# speedup vs baseline: 33.6739x; 2.2992x over previous
import functools

import jax
import jax.numpy as jnp
from jax import lax
from jax.experimental import pallas as pl
from jax.experimental.pallas import tpu as pltpu

CROP = 14
C = 256
LEVEL_HW = [(64, 64), (32, 32), (16, 16), (8, 8), (4, 4)]
HMAX, WMAX = 64, 64
EPS = 1e-7
YPAD = HMAX + 1
KX = sum(w for _, w in LEVEL_HW)
NCOL = YPAD * C


ROWPAD = 16
XOFFS = [0, 64, 96, 112, 120]


def _tc_body(nb_blk, y0_sm, wy0_sm, wy1_sm, sx0_ref, sx1_ref, wx0_ref,
             wx1_ref, p0_ref, p1_ref, p2_ref, p3_ref, p4_ref,
             out_ref, ta_ref, tab_ref):
    i = pl.program_id(0)
    mp = nb_blk * ROWPAD

    @pl.when(i == 0)
    def _build():
        for lvl, p_ref in enumerate((p0_ref, p1_ref, p2_ref, p3_ref, p4_ref)):
            hl, wl = LEVEL_HW[lvl]
            xo = XOFFS[lvl]
            for y in range(hl):
                tab_ref[pl.ds(xo, wl), pl.ds(y * C, C)] = p_ref[y]
            tab_ref[pl.ds(xo, wl), pl.ds(hl * C, (YPAD - hl) * C)] = (
                jnp.zeros((wl, (YPAD - hl) * C), jnp.float32))
    sx0 = sx0_ref[...]
    sx1 = sx1_ref[...]
    wx0 = wx0_ref[...]
    wx1 = wx1_ref[...]
    iota = lax.broadcasted_iota(jnp.int32, (mp, KX), 1)
    w_oh = (jnp.where(iota == sx0, wx0, 0.0)
            + jnp.where(iota == sx1, wx1, 0.0))
    ta_ref[...] = jnp.dot(w_oh, tab_ref[...],
                          preferred_element_type=jnp.float32)

    for b in range(nb_blk):
        for j in range(CROP):
            g = i * (nb_blk * CROP) + b * CROP + j
            y0 = y0_sm[g]
            ystart = pl.multiple_of(y0 * C, C)
            s = ta_ref[pl.ds(b * ROWPAD, CROP), pl.ds(ystart, 2 * C)]
            out_ref[0, b, j] = s[:, :C] * wy0_sm[g] + s[:, C:] * wy1_sm[g]


def _run(nb, bb, image_shape, boxes, p_list):
    img = image_shape.astype(jnp.float32)
    b = boxes[0]
    x1, y1, x2, y2 = b[:, 0], b[:, 1], b[:, 2], b[:, 3]
    w = x2 - x1
    h = y2 - y1
    size = jnp.sqrt(w * h)
    levels = jnp.floor(1.0 + jnp.log2(size / 224.0 + EPS))
    levels = jnp.clip(levels, 0.0, 4.0).astype(jnp.int32)
    Hs = jnp.array([hw[0] for hw in LEVEL_HW], dtype=jnp.float32)
    Ws = jnp.array([hw[1] for hw in LEVEL_HW], dtype=jnp.float32)
    fh = Hs[levels]
    fw = Ws[levels]
    y1n = y1 / img[1] * fh / (fh - 1.0)
    x1n = x1 / img[2] * fw / (fw - 1.0)
    y2n = (y2 / img[1] * fh - 1.0) / (fh - 1.0)
    x2n = (x2 / img[2] * fw - 1.0) / (fw - 1.0)
    i14 = jnp.arange(CROP, dtype=jnp.float32)
    ys = (y1n[:, None] * (fh[:, None] - 1.0)
          + i14[None, :] * (y2n - y1n)[:, None] * (fh[:, None] - 1.0) / (CROP - 1.0))
    xs = (x1n[:, None] * (fw[:, None] - 1.0)
          + i14[None, :] * (x2n - x1n)[:, None] * (fw[:, None] - 1.0) / (CROP - 1.0))
    valid_y = ((ys >= 0.0) & (ys <= fh[:, None] - 1.0)).astype(jnp.float32)
    valid_x = ((xs >= 0.0) & (xs <= fw[:, None] - 1.0)).astype(jnp.float32)
    y0f = jnp.floor(ys)
    x0f = jnp.floor(xs)
    ly = ys - y0f
    lx = xs - x0f
    y0i = jnp.clip(y0f, 0, HMAX - 1).astype(jnp.int32)
    x0i = jnp.clip(x0f, 0, WMAX - 1).astype(jnp.int32)
    x1i = jnp.clip(x0f + 1.0, 0, WMAX - 1).astype(jnp.int32)
    xoffv = jnp.array(XOFFS, dtype=jnp.int32)[levels]
    sx0 = xoffv[:, None] + x0i
    sx1 = xoffv[:, None] + x1i
    wy0 = (1.0 - ly) * valid_y
    wy1 = ly * valid_y
    wx0 = (1.0 - lx) * valid_x
    wx1 = lx * valid_x

    n_blk = nb // bb
    mp = bb * ROWPAD
    padrows = ((0, 0), (0, ROWPAD - CROP))
    coli = lambda a: jnp.pad(a, padrows, constant_values=-1).reshape(
        nb * ROWPAD, 1)
    colf = lambda a: jnp.pad(a, padrows).reshape(nb * ROWPAD, 1)

    grid_spec = pltpu.PrefetchScalarGridSpec(
        num_scalar_prefetch=3,
        grid=(n_blk,),
        in_specs=[
            pl.BlockSpec((mp, 1), lambda i, *_: (i, 0)),
            pl.BlockSpec((mp, 1), lambda i, *_: (i, 0)),
            pl.BlockSpec((mp, 1), lambda i, *_: (i, 0)),
            pl.BlockSpec((mp, 1), lambda i, *_: (i, 0)),
        ] + [
            pl.BlockSpec(
                (LEVEL_HW[l][0], LEVEL_HW[l][1], C),
                lambda i, *_: (0, 0, 0))
            for l in range(5)
        ],
        out_specs=pl.BlockSpec((1, bb, CROP, CROP, C),
                               lambda i, *_: (0, i, 0, 0, 0)),
        scratch_shapes=[pltpu.VMEM((mp, NCOL), jnp.float32),
                        pltpu.VMEM((KX, NCOL), jnp.float32)],
    )
    out5 = pl.pallas_call(
        functools.partial(_tc_body, bb),
        grid_spec=grid_spec,
        out_shape=jax.ShapeDtypeStruct((1, nb, CROP, CROP, C), jnp.float32),
    )(y0i.reshape(-1), wy0.reshape(-1), wy1.reshape(-1),
      coli(sx0), coli(sx1), colf(wx0), colf(wx1),
      p_list[0][0], p_list[1][0], p_list[2][0], p_list[3][0], p_list[4][0])
    return out5


def kernel(image_shape, boxes, scores, p0, p1, p2, p3, p4):
    del scores
    return _run(boxes.shape[1], 8, image_shape, boxes, (p0, p1, p2, p3, p4))
